# Initial kernel scaffold; baseline (speedup 1.0000x reference)
import jax
import jax.numpy as jnp
from jax.experimental import pallas as pl


def kernel(x, table):
    raise NotImplementedError("write your pallas kernel here")


# speedup vs baseline: 1.4673x; 1.4673x over previous
import functools

import jax
import jax.numpy as jnp
from jax import lax
from jax.experimental import pallas as pl
from jax.experimental.pallas import tpu as pltpu
from jax.experimental.pallas import tpu_sc as plsc

D_MODEL = 32
NUM_CORES = 2
NUM_SUBCORES = 16
NW = NUM_CORES * NUM_SUBCORES
GROUP = 128


@functools.partial(jax.jit, static_argnames=("groups_per_worker",))
def _gather_rows(idx, table, groups_per_worker):
    G = groups_per_worker
    mesh = plsc.VectorSubcoreMesh(core_axis_name="c", subcore_axis_name="s")

    @functools.partial(
        pl.kernel,
        out_type=jax.ShapeDtypeStruct((NW, G, GROUP, D_MODEL), jnp.float32),
        mesh=mesh,
        scratch_types=[
            pltpu.VMEM((G, GROUP), jnp.int32),
            pltpu.VMEM((2, GROUP, D_MODEL), jnp.float32),
            pltpu.SemaphoreType.DMA,
            pltpu.SemaphoreType.DMA,
        ],
        compiler_params=pltpu.CompilerParams(use_tc_tiling_on_sc=False),
    )
    def k(table_hbm, idx_hbm, out_hbm, idx_v, rows_v, gsem, ssem):
        wid = lax.axis_index("s") * NUM_CORES + lax.axis_index("c")
        pltpu.sync_copy(idx_hbm.at[wid], idx_v)
        pltpu.async_copy(table_hbm.at[idx_v.at[0]], rows_v.at[0], gsem)

        def step(g, carry):
            b = lax.rem(g, 2)
            cur = rows_v.at[b]
            pltpu.make_async_copy(table_hbm.at[idx_v.at[g]], cur, gsem).wait()

            @pl.when(g >= 1)
            def _():
                pltpu.make_async_copy(
                    rows_v.at[1 - b], out_hbm.at[wid, g], ssem
                ).wait()

            @pl.when(g + 1 < G)
            def _():
                pltpu.async_copy(
                    table_hbm.at[idx_v.at[g + 1]], rows_v.at[1 - b], gsem
                )

            pltpu.async_copy(cur, out_hbm.at[wid, g], ssem)
            return carry

        lax.fori_loop(0, G, step, 0)
        pltpu.make_async_copy(rows_v.at[0], out_hbm.at[wid, 0], ssem).wait()

    return k(table, idx)


def kernel(x, table):
    B_total = x.shape[0] * x.shape[1]
    chunk = NW * GROUP
    B_pad = ((B_total + chunk - 1) // chunk) * chunk
    G = B_pad // chunk
    xf = x.reshape(-1)
    if B_pad != B_total:
        xf = jnp.concatenate(
            [xf, jnp.zeros((B_pad - B_total,), dtype=xf.dtype)]
        )
    idx = xf.reshape(NW, G, GROUP)
    rows = _gather_rows(idx, table, G)
    rows = rows.reshape(B_pad, D_MODEL)[:B_total]
    return rows.reshape(x.shape[0], x.shape[1], D_MODEL)


# speedup vs baseline: 1.5758x; 1.0740x over previous
import functools

import jax
import jax.numpy as jnp
from jax import lax
from jax.experimental import pallas as pl
from jax.experimental.pallas import tpu as pltpu
from jax.experimental.pallas import tpu_sc as plsc

D_MODEL = 32
NUM_CORES = 2
NUM_SUBCORES = 16
NW = NUM_CORES * NUM_SUBCORES
GROUP = 128
NBUF = 8


@functools.partial(jax.jit, static_argnames=("groups_per_worker",))
def _gather_rows(idx, table, groups_per_worker):
    G = groups_per_worker
    mesh = plsc.VectorSubcoreMesh(core_axis_name="c", subcore_axis_name="s")

    @functools.partial(
        pl.kernel,
        out_type=jax.ShapeDtypeStruct((NW, G, GROUP, D_MODEL), jnp.float32),
        mesh=mesh,
        scratch_types=[
            pltpu.VMEM((G, GROUP), jnp.int32),
            pltpu.VMEM((NBUF, GROUP, D_MODEL), jnp.float32),
            pltpu.SemaphoreType.DMA,
            pltpu.SemaphoreType.DMA,
        ],
        compiler_params=pltpu.CompilerParams(use_tc_tiling_on_sc=False),
    )
    def k(table_hbm, idx_hbm, out_hbm, idx_v, rows_v, gsem, ssem):
        wid = lax.axis_index("s") * NUM_CORES + lax.axis_index("c")
        pltpu.sync_copy(idx_hbm.at[wid], idx_v)
        for b in range(min(NBUF, G)):
            pltpu.async_copy(table_hbm.at[idx_v.at[b]], rows_v.at[b], gsem)

        def step(g, carry):
            cur = rows_v.at[lax.rem(g, NBUF)]
            pltpu.make_async_copy(table_hbm.at[idx_v.at[g]], cur, gsem).wait()

            @pl.when(g >= 1)
            def _():
                pltpu.make_async_copy(
                    rows_v.at[0], out_hbm.at[wid, 0], ssem
                ).wait()

                @pl.when(g + NBUF - 1 < G)
                def _():
                    pltpu.async_copy(
                        table_hbm.at[idx_v.at[g + NBUF - 1]],
                        rows_v.at[lax.rem(g - 1, NBUF)],
                        gsem,
                    )

            pltpu.async_copy(cur, out_hbm.at[wid, g], ssem)
            return carry

        lax.fori_loop(0, G, step, 0)
        pltpu.make_async_copy(rows_v.at[0], out_hbm.at[wid, 0], ssem).wait()

    return k(table, idx)


def kernel(x, table):
    B_total = x.shape[0] * x.shape[1]
    chunk = NW * GROUP
    B_pad = ((B_total + chunk - 1) // chunk) * chunk
    G = B_pad // chunk
    xf = x.reshape(-1)
    if B_pad != B_total:
        xf = jnp.concatenate(
            [xf, jnp.zeros((B_pad - B_total,), dtype=xf.dtype)]
        )
    idx = xf.reshape(NW, G, GROUP)
    rows = _gather_rows(idx, table, G)
    rows = rows.reshape(B_pad, D_MODEL)[:B_total]
    return rows.reshape(x.shape[0], x.shape[1], D_MODEL)


# speedup vs baseline: 1.5817x; 1.0037x over previous
import functools

import jax
import jax.numpy as jnp
from jax import lax
from jax.experimental import pallas as pl
from jax.experimental.pallas import tpu as pltpu
from jax.experimental.pallas import tpu_sc as plsc

D_MODEL = 32
NUM_CORES = 2
NUM_SUBCORES = 16
NW = NUM_CORES * NUM_SUBCORES
GROUP = 256
NBUF = 8


@functools.partial(jax.jit, static_argnames=("groups_per_worker",))
def _gather_rows(idx, table, groups_per_worker):
    G = groups_per_worker
    mesh = plsc.VectorSubcoreMesh(core_axis_name="c", subcore_axis_name="s")

    @functools.partial(
        pl.kernel,
        out_type=jax.ShapeDtypeStruct((NW, G, GROUP, D_MODEL), jnp.float32),
        mesh=mesh,
        scratch_types=[
            pltpu.VMEM((G, GROUP), jnp.int32),
            pltpu.VMEM((NBUF, GROUP, D_MODEL), jnp.float32),
            pltpu.SemaphoreType.DMA,
            pltpu.SemaphoreType.DMA,
        ],
        compiler_params=pltpu.CompilerParams(use_tc_tiling_on_sc=False),
    )
    def k(table_hbm, idx_hbm, out_hbm, idx_v, rows_v, gsem, ssem):
        wid = lax.axis_index("s") * NUM_CORES + lax.axis_index("c")
        pltpu.sync_copy(idx_hbm.at[wid], idx_v)
        for b in range(min(NBUF, G)):
            pltpu.async_copy(table_hbm.at[idx_v.at[b]], rows_v.at[b], gsem)

        def step(g, carry):
            cur = rows_v.at[lax.rem(g, NBUF)]
            pltpu.make_async_copy(table_hbm.at[idx_v.at[g]], cur, gsem).wait()

            @pl.when(g >= 1)
            def _():
                pltpu.make_async_copy(
                    rows_v.at[0], out_hbm.at[wid, 0], ssem
                ).wait()

                @pl.when(g + NBUF - 1 < G)
                def _():
                    pltpu.async_copy(
                        table_hbm.at[idx_v.at[g + NBUF - 1]],
                        rows_v.at[lax.rem(g - 1, NBUF)],
                        gsem,
                    )

            pltpu.async_copy(cur, out_hbm.at[wid, g], ssem)
            return carry

        lax.fori_loop(0, G, step, 0)
        pltpu.make_async_copy(rows_v.at[0], out_hbm.at[wid, 0], ssem).wait()

    return k(table, idx)


def kernel(x, table):
    B_total = x.shape[0] * x.shape[1]
    chunk = NW * GROUP
    B_pad = ((B_total + chunk - 1) // chunk) * chunk
    G = B_pad // chunk
    xf = x.reshape(-1)
    if B_pad != B_total:
        xf = jnp.concatenate(
            [xf, jnp.zeros((B_pad - B_total,), dtype=xf.dtype)]
        )
    idx = xf.reshape(NW, G, GROUP)
    rows = _gather_rows(idx, table, G)
    rows = rows.reshape(B_pad, D_MODEL)[:B_total]
    return rows.reshape(x.shape[0], x.shape[1], D_MODEL)
